# baseline (speedup 1.0000x reference)
import jax
import jax.numpy as jnp
import numpy as np
from jax import lax
from jax.experimental import pallas as pl
from jax.experimental.pallas import tpu as pltpu
from jax.experimental.pallas import tpu_sc as plsc

B, T, D, A = 4, 8192, 768, 256
RINGS = 1
TB = 1024
NT = T // TB

NC, NS, L = 2, 16, 16
NW = NC * NS
TCUT = 0
STRIDE = (T - TCUT) // (NW // B)
CH = 64
NCHUNK = STRIDE // CH


def _neighbor_w():
    w = np.zeros((A, A), dtype=np.float32)
    for a in range(A):
        lo, hi = max(0, a - RINGS), min(A, a + RINGS + 1)
        w[a, lo:hi] = 1.0 / (hi - lo)
    return jnp.asarray(w)


def _pool_body(assign_ref, x_ref, sums_ref):
    tb = pl.program_id(1)
    a_ids = assign_ref[0, 0, :]
    rows = lax.broadcasted_iota(jnp.int32, (A, TB), 0)
    onehot_t = (rows == a_ids[None, :]).astype(jnp.bfloat16)
    x_blk = x_ref[0].astype(jnp.bfloat16)
    partial = lax.dot_general(
        onehot_t, x_blk, (((1,), (0,)), ((), ())),
        preferred_element_type=jnp.float32)

    @pl.when(tb == 0)
    def _():
        sums_ref[0] = partial

    @pl.when(tb != 0)
    def _():
        sums_ref[0] += partial


def _mid_body(gate_ref, hops_ref, assign_ref, sums_ref, wn_ref, wp_ref,
              scaled_ref):
    rows = lax.broadcasted_iota(jnp.int32, (A, TB), 0)
    counts = jnp.zeros((A,), jnp.float32)
    for t in range(NT):
        onehot = (rows == assign_ref[t, 0, :][None, :]).astype(jnp.float32)
        counts = counts + jnp.sum(onehot, axis=1)
    inv = 1.0 / jnp.maximum(counts, 1.0)
    g = jnp.tanh(gate_ref[0])
    wn = wn_ref[...]
    wp = wp_ref[...]
    nhops = jnp.maximum(1, hops_ref[0])
    for b in range(B):
        anchor = sums_ref[b] * inv[:, None]
        agg = lax.fori_loop(
            0, nhops,
            lambda _, a: jnp.dot(wn, a, preferred_element_type=jnp.float32),
            anchor)
        proj = lax.dot_general(
            agg, wp, (((1,), (1,)), ((), ())),
            preferred_element_type=jnp.float32)
        scaled_ref[b] = proj * g


def _bcast_tc_body(assign_ref, x_ref, scaled_ref, out_ref):
    a_ids = assign_ref[0, 0, :]
    cols = lax.broadcasted_iota(jnp.int32, (TB, A), 1)
    onehot = (cols == a_ids[:, None]).astype(jnp.bfloat16)
    fb = lax.dot_general(
        onehot, scaled_ref[0].astype(jnp.bfloat16), (((1,), (0,)), ((), ())),
        preferred_element_type=jnp.float32)
    out_ref[0] = x_ref[0] + fb


def _bcast_sc_body(x_ref, assign_ref, scaled_ref, out_ref, *scratch):
    idx = scratch[:NCHUNK]
    xbuf, gbuf = scratch[NCHUNK], scratch[NCHUNK + 1]
    c = lax.axis_index("c")
    s = lax.axis_index("s")
    wid = s * NC + c
    b = wid // (NW // B)
    lane = wid % (NW // B)
    tok0 = TCUT + lane * STRIDE
    row0 = b * T + tok0
    boff = b * A

    for j in range(NCHUNK):
        pltpu.sync_copy(assign_ref.at[(tok0 // CH) + j], idx[j])
        for k in range(CH // L):
            idx[j][pl.ds(k * L, L)] = idx[j][pl.ds(k * L, L)] + boff

    for j in range(NCHUNK):
        pltpu.sync_copy(x_ref.at[pl.ds(row0 + j * CH, CH)], xbuf)
        pltpu.sync_copy(scaled_ref.at[plsc.Indices(idx[j])], gbuf)

        def _add(r, _):
            for k in range(D // L):
                sl = pl.ds(k * L, L)
                xbuf[r, sl] = xbuf[r, sl] + gbuf[r, sl]
            return 0

        lax.fori_loop(0, CH, _add, 0)
        pltpu.sync_copy(xbuf, out_ref.at[pl.ds(row0 + j * CH, CH)])


def _bcast_sc(x_flat, assign_chunks, scaled_flat):
    mesh = plsc.VectorSubcoreMesh(core_axis_name="c", subcore_axis_name="s")
    kfun = pl.kernel(
        _bcast_sc_body,
        out_type=jax.ShapeDtypeStruct((B * T, D), jnp.float32),
        mesh=mesh,
        scratch_types=(
            [pltpu.VMEM((CH,), jnp.int32) for _ in range(NCHUNK)]
            + [
                pltpu.VMEM((CH, D), jnp.float32),
                pltpu.VMEM((CH, D), jnp.float32),
            ]
        ),
    )
    return kfun(x_flat, assign_chunks, scaled_flat)


def kernel(x, assign, W_proj, gate, hops):
    assign_i = assign.astype(jnp.int32)
    assign3 = assign_i.reshape(NT, 1, TB)

    sums = pl.pallas_call(
        _pool_body,
        grid=(B, NT),
        in_specs=[
            pl.BlockSpec((1, 1, TB), lambda b, t: (t, 0, 0)),
            pl.BlockSpec((1, TB, D), lambda b, t: (b, t, 0)),
        ],
        out_specs=pl.BlockSpec((1, A, D), lambda b, t: (b, 0, 0)),
        out_shape=jax.ShapeDtypeStruct((B, A, D), jnp.float32),
    )(assign3, x)

    wn = _neighbor_w()
    gate_s = jnp.reshape(jnp.asarray(gate, jnp.float32), (1,))
    hops_s = jnp.reshape(jnp.asarray(hops, jnp.int32), (1,))
    scaled = pl.pallas_call(
        _mid_body,
        in_specs=[
            pl.BlockSpec(memory_space=pltpu.SMEM),
            pl.BlockSpec(memory_space=pltpu.SMEM),
            pl.BlockSpec((NT, 1, TB), lambda: (0, 0, 0)),
            pl.BlockSpec((B, A, D), lambda: (0, 0, 0)),
            pl.BlockSpec((A, A), lambda: (0, 0)),
            pl.BlockSpec((D, D), lambda: (0, 0)),
        ],
        out_specs=pl.BlockSpec((B, A, D), lambda: (0, 0, 0)),
        out_shape=jax.ShapeDtypeStruct((B, A, D), jnp.float32),
    )(gate_s, hops_s, assign3, sums, wn, W_proj)

    out_sc = _bcast_sc(x.reshape(B * T, D),
                       assign_i.reshape(T // CH, CH),
                       scaled.reshape(B * A, D)).reshape(B, T, D)

    if TCUT > 0:
        ntc = TCUT // TB
        out_tc = pl.pallas_call(
            _bcast_tc_body,
            grid=(B, ntc),
            in_specs=[
                pl.BlockSpec((1, 1, TB), lambda b, t: (t, 0, 0)),
                pl.BlockSpec((1, TB, D), lambda b, t: (b, t, 0)),
                pl.BlockSpec((1, A, D), lambda b, t: (b, 0, 0)),
            ],
            out_specs=pl.BlockSpec((1, TB, D), lambda b, t: (b, t, 0)),
            out_shape=jax.ShapeDtypeStruct((B, TCUT, D), jnp.float32),
        )(assign3[:ntc], x[:, :TCUT], scaled)
        out = jnp.concatenate([out_tc, out_sc[:, TCUT:]], axis=1)
    else:
        out = out_sc

    return out


# speedup vs baseline: 1.2725x; 1.2725x over previous
import jax
import jax.numpy as jnp
import numpy as np
from jax import lax
from jax.experimental import pallas as pl
from jax.experimental.pallas import tpu as pltpu
from jax.experimental.pallas import tpu_sc as plsc

B, T, D, A = 4, 8192, 768, 256
RINGS = 1
TB = 1024
NT = T // TB

NC, NS, L = 2, 16, 16
NW = NC * NS
TCUT = 0
STRIDE = (T - TCUT) // (NW // B)
CH = 32
NCHUNK = STRIDE // CH


def _neighbor_w():
    w = np.zeros((A, A), dtype=np.float32)
    for a in range(A):
        lo, hi = max(0, a - RINGS), min(A, a + RINGS + 1)
        w[a, lo:hi] = 1.0 / (hi - lo)
    return jnp.asarray(w)


def _pool_body(assign_ref, x_ref, sums_ref, counts_ref):
    b = pl.program_id(0)
    tb = pl.program_id(1)
    a_ids = assign_ref[0, 0, :]
    rows = lax.broadcasted_iota(jnp.int32, (A, TB), 0)
    onehot_t = (rows == a_ids[None, :]).astype(jnp.bfloat16)
    x_blk = x_ref[0].astype(jnp.bfloat16)
    partial = lax.dot_general(
        onehot_t, x_blk, (((1,), (0,)), ((), ())),
        preferred_element_type=jnp.float32)

    @pl.when(tb == 0)
    def _():
        sums_ref[0] = partial

    @pl.when(tb != 0)
    def _():
        sums_ref[0] += partial

    cpart = jnp.sum((rows == a_ids[None, :]).astype(jnp.float32), axis=1)

    @pl.when((b == 0) & (tb == 0))
    def _():
        counts_ref[0, :] = cpart

    @pl.when((b == 0) & (tb != 0))
    def _():
        counts_ref[0, :] += cpart


def _mid_body(gate_ref, hops_ref, sums_ref, counts_ref, wn_ref, wp_ref,
              scaled_ref):
    inv = 1.0 / jnp.maximum(counts_ref[0, :], 1.0)
    g = jnp.tanh(gate_ref[0])
    wn = wn_ref[...]
    wp = wp_ref[...]
    nhops = jnp.maximum(1, hops_ref[0])
    for b in range(B):
        anchor = sums_ref[b] * inv[:, None]
        agg = lax.fori_loop(
            0, nhops,
            lambda _, a: jnp.dot(wn, a, preferred_element_type=jnp.float32),
            anchor)
        proj = lax.dot_general(
            agg, wp, (((1,), (1,)), ((), ())),
            preferred_element_type=jnp.float32)
        scaled_ref[b] = proj * g


def _bcast_tc_body(assign_ref, x_ref, scaled_ref, out_ref):
    a_ids = assign_ref[0, 0, :]
    cols = lax.broadcasted_iota(jnp.int32, (TB, A), 1)
    onehot = (cols == a_ids[:, None]).astype(jnp.bfloat16)
    fb = lax.dot_general(
        onehot, scaled_ref[0].astype(jnp.bfloat16), (((1,), (0,)), ((), ())),
        preferred_element_type=jnp.float32)
    out_ref[0] = x_ref[0] + fb


def _bcast_sc_body(x_ref, assign_ref, scaled_ref, out_ref, idx_all,
                   xb0, gb0, xb1, gb1, sx0, sg0, st0, sx1, sg1, st1):
    c = lax.axis_index("c")
    s = lax.axis_index("s")
    wid = s * NC + c
    b = wid // (NW // B)
    lane = wid % (NW // B)
    tok0 = TCUT + lane * STRIDE
    row0 = b * T + tok0
    boff = b * A
    bufs = ((xb0, gb0, sx0, sg0, st0), (xb1, gb1, sx1, sg1, st1))

    pltpu.sync_copy(assign_ref.at[pl.ds(tok0, STRIDE)], idx_all)

    def _off(k, _):
        sl = pl.ds(k * L, L)
        idx_all[sl] = idx_all[sl] + boff
        return 0

    lax.fori_loop(0, STRIDE // L, _off, 0)

    def issue_loads(j, slot):
        xb, gb, sx, sg, _ = bufs[slot]
        pltpu.async_copy(x_ref.at[pl.ds(row0 + j * CH, CH)], xb, sx)
        pltpu.async_copy(
            scaled_ref.at[plsc.Indices(idx_all.at[pl.ds(j * CH, CH)])],
            gb, sg)

    issue_loads(0, 0)
    issue_loads(1, 1)

    def chunk_pair(j2, _):
        j0 = 2 * j2
        for u in range(2):
            j = j0 + u
            xb, gb, sx, sg, st = bufs[u]
            pltpu.make_async_copy(
                x_ref.at[pl.ds(row0 + j * CH, CH)], xb, sx).wait()
            pltpu.make_async_copy(
                scaled_ref.at[plsc.Indices(idx_all.at[pl.ds(j * CH, CH)])],
                gb, sg).wait()

            def _add(r, _):
                for k in range(D // L):
                    sl = pl.ds(k * L, L)
                    plsc.addupdate(xb.at[r, sl], gb[r, sl])
                return 0

            lax.fori_loop(0, CH, _add, 0)
            dst = out_ref.at[pl.ds(row0 + j * CH, CH)]
            pltpu.async_copy(xb, dst, st)

            @pl.when(j + 2 < NCHUNK)
            def _():
                pltpu.make_async_copy(xb, dst, st).wait()
                jn = j + 2
                pltpu.async_copy(x_ref.at[pl.ds(row0 + jn * CH, CH)], xb, sx)
                pltpu.async_copy(
                    scaled_ref.at[
                        plsc.Indices(idx_all.at[pl.ds(jn * CH, CH)])],
                    gb, sg)

        return 0

    lax.fori_loop(0, NCHUNK // 2, chunk_pair, 0)

    for u in range(2):
        j = NCHUNK - 2 + u
        xb, _, _, _, st = bufs[u]
        pltpu.make_async_copy(
            xb, out_ref.at[pl.ds(row0 + j * CH, CH)], st).wait()


def _bcast_sc(x_flat, assign_flat, scaled_flat):
    mesh = plsc.VectorSubcoreMesh(core_axis_name="c", subcore_axis_name="s")
    kfun = pl.kernel(
        _bcast_sc_body,
        out_type=jax.ShapeDtypeStruct((B * T, D), jnp.float32),
        mesh=mesh,
        scratch_types=[
            pltpu.VMEM((STRIDE,), jnp.int32),
            pltpu.VMEM((CH, D), jnp.float32),
            pltpu.VMEM((CH, D), jnp.float32),
            pltpu.VMEM((CH, D), jnp.float32),
            pltpu.VMEM((CH, D), jnp.float32),
            pltpu.SemaphoreType.DMA,
            pltpu.SemaphoreType.DMA,
            pltpu.SemaphoreType.DMA,
            pltpu.SemaphoreType.DMA,
            pltpu.SemaphoreType.DMA,
            pltpu.SemaphoreType.DMA,
        ],
    )
    return kfun(x_flat, assign_flat, scaled_flat)


def kernel(x, assign, W_proj, gate, hops):
    assign_i = assign.astype(jnp.int32)
    assign3 = assign_i.reshape(NT, 1, TB)

    sums, counts = pl.pallas_call(
        _pool_body,
        grid=(B, NT),
        in_specs=[
            pl.BlockSpec((1, 1, TB), lambda b, t: (t, 0, 0)),
            pl.BlockSpec((1, TB, D), lambda b, t: (b, t, 0)),
        ],
        out_specs=[
            pl.BlockSpec((1, A, D), lambda b, t: (b, 0, 0)),
            pl.BlockSpec((1, A), lambda b, t: (0, 0)),
        ],
        out_shape=[
            jax.ShapeDtypeStruct((B, A, D), jnp.float32),
            jax.ShapeDtypeStruct((1, A), jnp.float32),
        ],
    )(assign3, x)

    wn = _neighbor_w()
    gate_s = jnp.reshape(jnp.asarray(gate, jnp.float32), (1,))
    hops_s = jnp.reshape(jnp.asarray(hops, jnp.int32), (1,))
    scaled = pl.pallas_call(
        _mid_body,
        in_specs=[
            pl.BlockSpec(memory_space=pltpu.SMEM),
            pl.BlockSpec(memory_space=pltpu.SMEM),
            pl.BlockSpec((B, A, D), lambda: (0, 0, 0)),
            pl.BlockSpec((1, A), lambda: (0, 0)),
            pl.BlockSpec((A, A), lambda: (0, 0)),
            pl.BlockSpec((D, D), lambda: (0, 0)),
        ],
        out_specs=pl.BlockSpec((B, A, D), lambda: (0, 0, 0)),
        out_shape=jax.ShapeDtypeStruct((B, A, D), jnp.float32),
    )(gate_s, hops_s, sums, counts, wn, W_proj)

    out_sc = _bcast_sc(x.reshape(B * T, D), assign_i,
                       scaled.reshape(B * A, D)).reshape(B, T, D)

    if TCUT > 0:
        ntc = TCUT // TB
        out_tc = pl.pallas_call(
            _bcast_tc_body,
            grid=(B, ntc),
            in_specs=[
                pl.BlockSpec((1, 1, TB), lambda b, t: (t, 0, 0)),
                pl.BlockSpec((1, TB, D), lambda b, t: (b, t, 0)),
                pl.BlockSpec((1, A, D), lambda b, t: (b, 0, 0)),
            ],
            out_specs=pl.BlockSpec((1, TB, D), lambda b, t: (b, t, 0)),
            out_shape=jax.ShapeDtypeStruct((B, TCUT, D), jnp.float32),
        )(assign3[:ntc], x[:, :TCUT], scaled)
        out = jnp.concatenate([out_tc, out_sc[:, TCUT:]], axis=1)
    else:
        out = out_sc

    return out
